# Initial kernel scaffold; baseline (speedup 1.0000x reference)
import jax
import jax.numpy as jnp
from jax.experimental import pallas as pl


def kernel(inputs, W1, b1, W2, b2, a_lsh, b_lsh):
    raise NotImplementedError("write your pallas kernel here")


# speedup vs baseline: 2.5065x; 2.5065x over previous
import functools

import jax
import jax.numpy as jnp
import numpy as np
from jax import lax
from jax.experimental import pallas as pl
from jax.experimental.pallas import tpu as pltpu
from jax.experimental.pallas import tpu_sc as plsc

_N = 16384
_HID = 256
_OUT = 128
_NH = 16
_NB = 1024

_MIX = np.zeros((1, _OUT), dtype=np.int32)
_MIX[0, :_NH] = np.array(
    [73856093, 19349663, 83492791, 49979687, 67867967, 86028121,
     15485863, 32452843, 49979693, 67867979, 86028157, 15485867,
     2654435761 % (2**31 - 1), 40503, 2246822519 % (2**31 - 1),
     3266489917 % (2**31 - 1)], dtype=np.int32)


_BLK = 2048
_GRID = _N // _BLK


def _tc_body(x_ref, w1_ref, b1_ref, w2_ref, b2_ref, al_ref, bl_ref, pr_ref,
             out_ref, bkt_ref):
    x = x_ref[...]
    h = lax.dot_general(x, w1_ref[...], (((1,), (0,)), ((), ())),
                        preferred_element_type=jnp.float32)
    h = jnp.maximum(h + b1_ref[...], 0.0)
    out = lax.dot_general(h, w2_ref[...], (((1,), (0,)), ((), ())),
                          preferred_element_type=jnp.float32)
    out = out + b2_ref[...]
    out_ref[...] = out
    proj = lax.dot_general(out, al_ref[...], (((1,), (0,)), ((), ())),
                           preferred_element_type=jnp.float32)
    proj = proj + bl_ref[...]
    hcodes = jnp.floor(proj * 0.25).astype(jnp.int32)
    mixed = jnp.sum(hcodes * pr_ref[...], axis=1, keepdims=True)
    bkt_ref[...] = lax.bitwise_and(mixed, 1023)


_tc_call = pl.pallas_call(
    _tc_body,
    grid=(_GRID,),
    in_specs=[
        pl.BlockSpec((_BLK, _HID), lambda i: (i, 0)),
        pl.BlockSpec((_HID, _HID), lambda i: (0, 0)),
        pl.BlockSpec((1, _HID), lambda i: (0, 0)),
        pl.BlockSpec((_HID, _OUT), lambda i: (0, 0)),
        pl.BlockSpec((1, _OUT), lambda i: (0, 0)),
        pl.BlockSpec((_OUT, _OUT), lambda i: (0, 0)),
        pl.BlockSpec((1, _OUT), lambda i: (0, 0)),
        pl.BlockSpec((1, _OUT), lambda i: (0, 0)),
    ],
    out_specs=[
        pl.BlockSpec((_BLK, _OUT), lambda i: (i, 0)),
        pl.BlockSpec((_BLK, 1), lambda i: (i, 0)),
    ],
    out_shape=[
        jax.ShapeDtypeStruct((_N, _OUT), jnp.float32),
        jax.ShapeDtypeStruct((_N, 1), jnp.int32),
    ],
)


_NS = 16
_PT = _N // _NS
_BT = _NB // _NS
_C = 128
_NCH = _PT // _C


@functools.cache
def _build_sc_call():
  mesh = plsc.VectorSubcoreMesh(core_axis_name="c", subcore_axis_name="s",
                                num_cores=1, num_subcores=_NS)

  @functools.partial(
      pl.kernel,
      out_type=(jax.ShapeDtypeStruct((_NB,), jnp.float32),
                jax.ShapeDtypeStruct((_NB, _OUT), jnp.float32)),
      mesh=mesh,
      compiler_params=pltpu.CompilerParams(needs_layout_passes=False),
      scratch_types=[
          pltpu.VMEM((_C, _OUT), jnp.float32),
          pltpu.VMEM((_C, _OUT), jnp.float32),
          pltpu.VMEM((_C,), jnp.int32),
          pltpu.VMEM((_C,), jnp.float32),
          pltpu.VMEM((_C,), jnp.float32),
          pltpu.VMEM((_BT, _OUT), jnp.float32),
          pltpu.VMEM((_BT,), jnp.float32),
          pltpu.VMEM((16, 17), jnp.float32),
          pltpu.VMEM_SHARED((_NB, _OUT), jnp.float32),
          pltpu.VMEM_SHARED((_NB,), jnp.float32),
          pltpu.VMEM_SHARED((_NB,), jnp.float32),
      ],
  )
  def _sc_call(out_hbm, bkt_hbm, qs_hbm, means_hbm,
               rows_v, mrows_v, idx_v, dist_v, ones_v, work_v, cnt_v,
               tbuf_v, sums_sh, cnt_sh, qs_sh):
    t = lax.axis_index("s")
    base = t * _PT
    brow = t * _BT

    zeros16 = jnp.zeros((16,), jnp.float32)
    for g in range(8):
      ones_v[pl.ds(g * 16, 16)] = jnp.ones((16,), jnp.float32)
      dist_v[pl.ds(g * 16, 16)] = zeros16

    def _zrow(r, carry):
      for g in range(8):
        work_v[r, pl.ds(g * 16, 16)] = zeros16
      return carry
    lax.fori_loop(0, _BT, _zrow, 0)

    pltpu.sync_copy(work_v, sums_sh.at[pl.ds(brow, _BT)])
    pltpu.sync_copy(dist_v.at[pl.ds(0, _BT)], cnt_sh.at[pl.ds(brow, _BT)])
    pltpu.sync_copy(dist_v.at[pl.ds(0, _BT)], qs_sh.at[pl.ds(brow, _BT)])
    plsc.subcore_barrier()

    def _p1(k, carry):
      off = base + k * _C
      pltpu.sync_copy(out_hbm.at[pl.ds(off, _C)], rows_v)
      pltpu.sync_copy(bkt_hbm.at[pl.ds(off, _C)], idx_v)
      pltpu.sync_copy(rows_v, sums_sh.at[idx_v], add=True)
      pltpu.sync_copy(ones_v, cnt_sh.at[idx_v], add=True)
      return carry
    lax.fori_loop(0, _NCH, _p1, 0)
    plsc.subcore_barrier()

    pltpu.sync_copy(sums_sh.at[pl.ds(brow, _BT)], work_v)
    pltpu.sync_copy(cnt_sh.at[pl.ds(brow, _BT)], cnt_v)

    for rg in range(_BT // 16):
      cv = cnt_v[pl.ds(rg * 16, 16)]
      inv = 1.0 / jnp.maximum(cv, 1.0)
      for j in range(16):
        ivec = lax.broadcast_in_dim(inv[j], (16,), ())
        r = rg * 16 + j
        for g in range(8):
          sl = pl.ds(g * 16, 16)
          work_v[r, sl] = work_v[r, sl] * ivec
    pltpu.sync_copy(work_v, means_hbm.at[pl.ds(brow, _BT)])
    plsc.subcore_barrier()

    def _p3(k, carry):
      off = base + k * _C
      pltpu.sync_copy(out_hbm.at[pl.ds(off, _C)], rows_v)
      pltpu.sync_copy(bkt_hbm.at[pl.ds(off, _C)], idx_v)
      pltpu.sync_copy(means_hbm.at[idx_v], mrows_v)

      lid = lax.iota(jnp.int32, 16)

      def _pgrp(pg, c2):
        for j in range(16):
          r = pg * 16 + j
          acc = jnp.zeros((16,), jnp.float32)
          for g in range(8):
            sl = pl.ds(g * 16, 16)
            d = rows_v[r, sl] - mrows_v[r, sl]
            acc = acc + d * d
          tbuf_v[j, pl.ds(0, 16)] = acc
        dvec = jnp.zeros((16,), jnp.float32)
        for dcol in range(16):
          col = jnp.full((16,), dcol, jnp.int32)
          dvec = dvec + plsc.load_gather(tbuf_v, [lid, col])
        d2 = dvec + 1e-12
        i = lax.bitcast_convert_type(d2, jnp.int32)
        i = 0x5F3759DF - lax.shift_right_logical(i, 1)
        y = lax.bitcast_convert_type(i, jnp.float32)
        y = y * (1.5 - 0.5 * d2 * y * y)
        y = y * (1.5 - 0.5 * d2 * y * y)
        y = y * (1.5 - 0.5 * d2 * y * y)
        dist_v[pl.ds(pg * 16, 16)] = d2 * y
        return c2
      lax.fori_loop(0, _C // 16, _pgrp, 0)

      pltpu.sync_copy(dist_v, qs_sh.at[idx_v], add=True)
      return carry
    lax.fori_loop(0, _NCH, _p3, 0)
    plsc.subcore_barrier()

    pltpu.sync_copy(qs_sh.at[pl.ds(brow, _BT)], dist_v.at[pl.ds(0, _BT)])
    pltpu.sync_copy(dist_v.at[pl.ds(0, _BT)], qs_hbm.at[pl.ds(brow, _BT)])

  return _sc_call


def kernel(inputs, W1, b1, W2, b2, a_lsh, b_lsh):
    a_pad = jnp.pad(a_lsh, ((0, 0), (0, _OUT - _NH)))
    bl_pad = jnp.pad(b_lsh, (0, _OUT - _NH)).reshape(1, _OUT)
    mix = jnp.asarray(_MIX)
    out, bkt2 = _tc_call(inputs, W1, b1.reshape(1, _HID), W2,
                         b2.reshape(1, _OUT), a_pad, bl_pad, mix)
    bkt = bkt2.reshape(_N)
    q_s, _ = _build_sc_call()(out, bkt)
    return q_s
